# baseline (speedup 1.0000x reference)
import functools

import jax
import jax.numpy as jnp
from jax import lax
from jax.experimental import pallas as pl
from jax.experimental.pallas import tpu as pltpu
from jax.experimental.pallas import tpu_sc as plsc


def kernel(out0, out1, out2, out3, comp_id):
    B, D = out0.shape
    info = plsc.get_sparse_core_info()
    NC, NS, L = info.num_cores, info.num_subcores, info.num_lanes
    NW = NC * NS
    bpw = B // NW
    ngrp = bpw // L
    cid1d = comp_id.reshape(B)

    mesh = plsc.VectorSubcoreMesh(core_axis_name="c", subcore_axis_name="s")

    @functools.partial(
        pl.kernel,
        mesh=mesh,
        out_type=jax.ShapeDtypeStruct((B, D), jnp.float32),
        scratch_types=[
            pltpu.VMEM((bpw,), jnp.int32),
            pltpu.VMEM((bpw + L,), jnp.int32),
            pltpu.VMEM((bpw + L,), jnp.int32),
            pltpu.VMEM((bpw + L,), jnp.int32),
            pltpu.VMEM((bpw + L,), jnp.int32),
            pltpu.VMEM((bpw + 4 * L, D), jnp.float32),
            pltpu.SemaphoreType.DMA,
            pltpu.SemaphoreType.DMA,
        ],
        compiler_params=pltpu.CompilerParams(needs_layout_passes=False),
    )
    def run(o0, o1, o2, o3, cid_hbm, out_hbm,
            cid_v, il0, il1, il2, il3, rows_v, gsem, ssem):
        tables = (o0, o1, o2, o3)
        ilists = (il0, il1, il2, il3)
        wid = lax.axis_index("s") * NC + lax.axis_index("c")
        base = wid * bpw
        pltpu.sync_copy(cid_hbm.at[pl.ds(base, bpw)], cid_v)

        def grp_body(g, carry):
            cnts, lasts = carry
            cid16 = cid_v[pl.ds(g * L, L)]
            rows16 = base + g * L + lax.iota(jnp.int32, L)
            new_cnts, new_lasts = [], []
            for k in range(4):
                mask = cid16 == k
                m32 = jnp.where(mask, 1, 0)
                inc = plsc.cumsum(m32)
                pos = cnts[k] + (inc - m32)
                plsc.store_scatter(ilists[k], [pos], rows16, mask=mask)
                npop = jnp.max(inc)
                lk = jnp.max(jnp.where(mask, rows16, -1))
                new_cnts.append(cnts[k] + npop)
                new_lasts.append(jnp.maximum(lasts[k], lk))
            return tuple(new_cnts), tuple(new_lasts)

        zero = jnp.int32(0)
        neg1 = jnp.int32(-1)
        cnts, lasts = lax.fori_loop(
            0, ngrp, grp_body,
            ((zero, zero, zero, zero), (neg1, neg1, neg1, neg1)))

        for k in range(4):
            ilists[k][pl.ds(cnts[k], L)] = jnp.full((L,), lasts[k], jnp.int32)

        nch = tuple((cnts[k] + (L - 1)) // L for k in range(4))
        offs = (zero, nch[0] * L, (nch[0] + nch[1]) * L,
                (nch[0] + nch[1] + nch[2]) * L)

        for k in range(4):
            def g_body(j, _, k=k):
                idx16 = ilists[k][pl.ds(j * L, L)]
                pltpu.make_async_copy(
                    tables[k].at[idx16],
                    rows_v.at[pl.ds(offs[k] + j * L, L), :],
                    gsem).start()
                return 0
            lax.fori_loop(0, nch[k], g_body, 0)

        for k in range(4):
            def g_wait(j, _, k=k):
                idx16 = ilists[k][pl.ds(j * L, L)]
                pltpu.make_async_copy(
                    tables[k].at[idx16],
                    rows_v.at[pl.ds(offs[k] + j * L, L), :],
                    gsem).wait()
                return 0
            lax.fori_loop(0, nch[k], g_wait, 0)

        for k in range(4):
            def s_body(j, _, k=k):
                idx16 = ilists[k][pl.ds(j * L, L)]
                pltpu.make_async_copy(
                    rows_v.at[pl.ds(offs[k] + j * L, L), :],
                    out_hbm.at[idx16],
                    ssem).start()
                return 0
            lax.fori_loop(0, nch[k], s_body, 0)

        for k in range(4):
            def s_wait(j, _, k=k):
                idx16 = ilists[k][pl.ds(j * L, L)]
                pltpu.make_async_copy(
                    rows_v.at[pl.ds(offs[k] + j * L, L), :],
                    out_hbm.at[idx16],
                    ssem).wait()
                return 0
            lax.fori_loop(0, nch[k], s_wait, 0)

    return run(out0, out1, out2, out3, cid1d)
